# Initial kernel scaffold; baseline (speedup 1.0000x reference)
import jax
import jax.numpy as jnp
from jax.experimental import pallas as pl


def kernel(x, y, z, d):
    raise NotImplementedError("write your pallas kernel here")


# speedup vs baseline: 23.1104x; 23.1104x over previous
import jax
import jax.numpy as jnp
from jax.experimental import pallas as pl
from jax.experimental.pallas import tpu as pltpu

_BIG_I32 = 2**30
_NEG_INF = float("-inf")
_POS_INF = float("inf")


def _topk_lane(blk, k, largest=True):
    lane = jax.lax.broadcasted_iota(jnp.int32, blk.shape, 1)
    cur = blk
    vs, js = [], []
    for _ in range(k):
        if largest:
            m = jnp.max(cur, axis=1, keepdims=True)
        else:
            m = jnp.min(cur, axis=1, keepdims=True)
        j = jnp.min(jnp.where(cur == m, lane, _BIG_I32), axis=1, keepdims=True)
        vs.append(m)
        js.append(j)
        cur = jnp.where(lane == j, _NEG_INF if largest else _POS_INF, cur)
    return vs, js


def _topk_sublane(blk, k, row0=0):
    rows = jax.lax.broadcasted_iota(jnp.int32, blk.shape, 0) + row0
    cur = blk
    vs, js = [], []
    for _ in range(k):
        m = jnp.max(cur, axis=0, keepdims=True)
        j = jnp.min(jnp.where(cur == m, rows, _BIG_I32), axis=0, keepdims=True)
        vs.append(m)
        js.append(j)
        cur = jnp.where(rows == j, _NEG_INF, cur)
    return vs, js


def _x_body(x_ref, v_ref, i_ref):
    vs, js = _topk_lane(x_ref[...], 4, largest=True)
    v_ref[...] = jnp.concatenate(vs, axis=1)
    i_ref[...] = jnp.concatenate(js, axis=1)


def _topk_x(x):
    rows, n = x.shape
    blk_r = 16
    return pl.pallas_call(
        _x_body,
        grid=(rows // blk_r,),
        in_specs=[pl.BlockSpec((blk_r, n), lambda g: (g, 0))],
        out_specs=[
            pl.BlockSpec((blk_r, 4), lambda g: (g, 0)),
            pl.BlockSpec((blk_r, 4), lambda g: (g, 0)),
        ],
        out_shape=[
            jax.ShapeDtypeStruct((rows, 4), jnp.float32),
            jax.ShapeDtypeStruct((rows, 4), jnp.int32),
        ],
    )(x)


def _y_body(y_ref, y1_ref, i1_ref, y2_ref, i2_ref):
    g = pl.program_id(0)
    blk = y_ref[...]
    blk_r = blk.shape[0]
    cols = jax.lax.broadcasted_iota(jnp.int32, blk.shape, 1)

    (m1, m2), (j1, j2) = _topk_sublane(blk, 2, row0=g * blk_r)

    @pl.when(g == 0)
    def _():
        y1_ref[...] = jnp.concatenate([m1, m2], axis=0)
        i1_ref[...] = jnp.concatenate([j1, j2], axis=0)

    @pl.when(g > 0)
    def _():
        v1 = y1_ref[0:1, :]
        v2 = y1_ref[1:2, :]
        p1 = i1_ref[0:1, :]
        p2 = i1_ref[1:2, :]
        take1 = m1 > v1
        a_v = jnp.where(take1, v1, v2)
        a_i = jnp.where(take1, p1, p2)
        b_v = jnp.where(take1, m2, m1)
        b_i = jnp.where(take1, j2, j1)
        take2 = b_v > a_v
        y1_ref[...] = jnp.concatenate(
            [jnp.where(take1, m1, v1), jnp.where(take2, b_v, a_v)], axis=0)
        i1_ref[...] = jnp.concatenate(
            [jnp.where(take1, j1, p1), jnp.where(take2, b_i, a_i)], axis=0)

    n1 = jnp.min(blk, axis=1, keepdims=True)
    k1 = jnp.min(jnp.where(blk == n1, cols, _BIG_I32), axis=1, keepdims=True)
    masked = jnp.where(cols == k1, _POS_INF, blk)
    n2 = jnp.min(masked, axis=1, keepdims=True)
    k2 = jnp.min(jnp.where(masked == n2, cols, _BIG_I32), axis=1, keepdims=True)
    y2_ref[...] = jnp.concatenate([n1, n2], axis=1)
    i2_ref[...] = jnp.concatenate([k1, k2], axis=1)


def _topk_y(y):
    rows, n = y.shape
    blk_r = 256
    grid = rows // blk_r
    return pl.pallas_call(
        _y_body,
        grid=(grid,),
        in_specs=[pl.BlockSpec((blk_r, n), lambda g: (g, 0))],
        out_specs=[
            pl.BlockSpec((2, n), lambda g: (0, 0)),
            pl.BlockSpec((2, n), lambda g: (0, 0)),
            pl.BlockSpec((blk_r, 2), lambda g: (g, 0)),
            pl.BlockSpec((blk_r, 2), lambda g: (g, 0)),
        ],
        out_shape=[
            jax.ShapeDtypeStruct((2, n), jnp.float32),
            jax.ShapeDtypeStruct((2, n), jnp.int32),
            jax.ShapeDtypeStruct((rows, 2), jnp.float32),
            jax.ShapeDtypeStruct((rows, 2), jnp.int32),
        ],
    )(y)


def _z_body(z_ref, i4_ref, z1_ref, i5_ref, i3_ref, v1_s, v2_s, p1_s, p2_s):
    g = pl.program_id(0)
    blk = z_ref[0]

    _, js = _topk_sublane(blk, 3)
    i4_ref[0] = jnp.concatenate(js, axis=0)

    vs, js = _topk_lane(blk, 1, largest=True)
    z1_ref[0] = vs[0]
    i5_ref[0] = js[0]

    @pl.when(g == 0)
    def _():
        v1_s[...] = blk
        p1_s[...] = jnp.zeros_like(p1_s)
        v2_s[...] = jnp.full_like(v2_s, _NEG_INF)
        p2_s[...] = jnp.zeros_like(p2_s)

    @pl.when(g > 0)
    def _():
        v1 = v1_s[...]
        v2 = v2_s[...]
        p1 = p1_s[...]
        p2 = p2_s[...]
        is1 = blk > v1
        is2 = jnp.logical_and(jnp.logical_not(is1), blk > v2)
        v2_s[...] = jnp.where(is1, v1, jnp.where(is2, blk, v2))
        p2_s[...] = jnp.where(is1, p1, jnp.where(is2, g, p2))
        v1_s[...] = jnp.where(is1, blk, v1)
        p1_s[...] = jnp.where(is1, g, p1)

    @pl.when(g == pl.num_programs(0) - 1)
    def _():
        i3_ref[0] = p1_s[...]
        i3_ref[1] = p2_s[...]


def _topk_z(z):
    a0, a1, a2 = z.shape
    return pl.pallas_call(
        _z_body,
        grid=(a0,),
        in_specs=[pl.BlockSpec((1, a1, a2), lambda g: (g, 0, 0))],
        out_specs=[
            pl.BlockSpec((1, 3, a2), lambda g: (g, 0, 0)),
            pl.BlockSpec((1, a1, 1), lambda g: (g, 0, 0)),
            pl.BlockSpec((1, a1, 1), lambda g: (g, 0, 0)),
            pl.BlockSpec((2, a1, a2), lambda g: (0, 0, 0)),
        ],
        out_shape=[
            jax.ShapeDtypeStruct((a0, 3, a2), jnp.int32),
            jax.ShapeDtypeStruct((a0, a1, 1), jnp.float32),
            jax.ShapeDtypeStruct((a0, a1, 1), jnp.int32),
            jax.ShapeDtypeStruct((2, a1, a2), jnp.int32),
        ],
        scratch_shapes=[
            pltpu.VMEM((a1, a2), jnp.float32),
            pltpu.VMEM((a1, a2), jnp.float32),
            pltpu.VMEM((a1, a2), jnp.int32),
            pltpu.VMEM((a1, a2), jnp.int32),
        ],
    )(z)


def _d_body(d_ref, d2_ref, i8_ref, d3_ref, i9_ref):
    blk = d_ref[0]

    vs, js = _topk_sublane(blk, 2)
    d2_ref[0] = jnp.concatenate(vs, axis=0)
    i8_ref[0] = jnp.concatenate(js, axis=0)

    vs, js = _topk_lane(blk, 2, largest=True)
    d3_ref[0] = jnp.concatenate(vs, axis=1)
    i9_ref[0] = jnp.concatenate(js, axis=1)


def _topk_d(dr):
    b, a1, a2 = dr.shape
    return pl.pallas_call(
        _d_body,
        grid=(b,),
        in_specs=[pl.BlockSpec((1, a1, a2), lambda g: (g, 0, 0))],
        out_specs=[
            pl.BlockSpec((1, 2, a2), lambda g: (g, 0, 0)),
            pl.BlockSpec((1, 2, a2), lambda g: (g, 0, 0)),
            pl.BlockSpec((1, a1, 2), lambda g: (g, 0, 0)),
            pl.BlockSpec((1, a1, 2), lambda g: (g, 0, 0)),
        ],
        out_shape=[
            jax.ShapeDtypeStruct((b, 2, a2), jnp.float32),
            jax.ShapeDtypeStruct((b, 2, a2), jnp.int32),
            jax.ShapeDtypeStruct((b, a1, 2), jnp.float32),
            jax.ShapeDtypeStruct((b, a1, 2), jnp.int32),
        ],
    )(dr)


def kernel(x, y, z, d):
    x0, i0 = _topk_x(x)
    y1, i1, y2, i2 = _topk_y(y)
    i4, z1, i5, i3 = _topk_z(z)

    b0, b1, a1, a2 = d.shape
    d2, i8, d3, i9 = _topk_d(d.reshape(b0 * b1, a1, a2))
    d2 = d2.reshape(b0, b1, 2, a2)
    i8 = i8.reshape(b0, b1, 2, a2)
    d3 = d3.reshape(b0, b1, a1, 2)
    i9 = i9.reshape(b0, b1, a1, 2)

    return (x0, y1, y2, i0, i1, i2, z1, i3, i4, i5, d2, d3, i8, i9)


# speedup vs baseline: 28.2861x; 1.2240x over previous
import jax
import jax.numpy as jnp
from jax.experimental import pallas as pl
from jax.experimental.pallas import tpu as pltpu

_BIG_I32 = 2**30
_NEG_INF = float("-inf")
_POS_INF = float("inf")


def _topk_axis(blk, k, axis, largest=True, idx_base=0):
    idx = jax.lax.broadcasted_iota(jnp.int32, blk.shape, axis) + idx_base
    fill = _NEG_INF if largest else _POS_INF
    cur = blk
    vs, js = [], []
    for t in range(k):
        if largest:
            m = jnp.max(cur, axis=axis, keepdims=True)
        else:
            m = jnp.min(cur, axis=axis, keepdims=True)
        j = jnp.min(jnp.where(cur == m, idx, _BIG_I32), axis=axis, keepdims=True)
        vs.append(m)
        js.append(j)
        if t + 1 < k:
            cur = jnp.where(idx == j, fill, cur)
    return vs, js


def _x_body(x_ref, v_ref, i_ref):
    vs, js = _topk_axis(x_ref[...], 4, 1, largest=True)
    v_ref[...] = jnp.concatenate(vs, axis=1)
    i_ref[...] = jnp.concatenate(js, axis=1)


def _topk_x(x):
    rows, n = x.shape
    blk_r = 16
    return pl.pallas_call(
        _x_body,
        grid=(rows // blk_r,),
        in_specs=[pl.BlockSpec((blk_r, n), lambda g: (g, 0))],
        out_specs=[
            pl.BlockSpec((blk_r, 4), lambda g: (g, 0)),
            pl.BlockSpec((blk_r, 4), lambda g: (g, 0)),
        ],
        out_shape=[
            jax.ShapeDtypeStruct((rows, 4), jnp.float32),
            jax.ShapeDtypeStruct((rows, 4), jnp.int32),
        ],
    )(x)


def _y_body(y_ref, y1_ref, i1_ref, y2_ref, i2_ref):
    g = pl.program_id(0)
    blk = y_ref[...]
    blk_r = blk.shape[0]

    (m1, m2), (j1, j2) = _topk_axis(blk, 2, 0, largest=True, idx_base=g * blk_r)

    @pl.when(g == 0)
    def _():
        y1_ref[...] = jnp.concatenate([m1, m2], axis=0)
        i1_ref[...] = jnp.concatenate([j1, j2], axis=0)

    @pl.when(g > 0)
    def _():
        v1 = y1_ref[0:1, :]
        v2 = y1_ref[1:2, :]
        p1 = i1_ref[0:1, :]
        p2 = i1_ref[1:2, :]
        take1 = m1 > v1
        a_v = jnp.where(take1, v1, v2)
        a_i = jnp.where(take1, p1, p2)
        b_v = jnp.where(take1, m2, m1)
        b_i = jnp.where(take1, j2, j1)
        take2 = b_v > a_v
        y1_ref[...] = jnp.concatenate(
            [jnp.where(take1, m1, v1), jnp.where(take2, b_v, a_v)], axis=0)
        i1_ref[...] = jnp.concatenate(
            [jnp.where(take1, j1, p1), jnp.where(take2, b_i, a_i)], axis=0)

    (n1, n2), (k1, k2) = _topk_axis(blk, 2, 1, largest=False)
    y2_ref[...] = jnp.concatenate([n1, n2], axis=1)
    i2_ref[...] = jnp.concatenate([k1, k2], axis=1)


def _topk_y(y):
    rows, n = y.shape
    blk_r = 256
    grid = rows // blk_r
    return pl.pallas_call(
        _y_body,
        grid=(grid,),
        in_specs=[pl.BlockSpec((blk_r, n), lambda g: (g, 0))],
        out_specs=[
            pl.BlockSpec((2, n), lambda g: (0, 0)),
            pl.BlockSpec((2, n), lambda g: (0, 0)),
            pl.BlockSpec((blk_r, 2), lambda g: (g, 0)),
            pl.BlockSpec((blk_r, 2), lambda g: (g, 0)),
        ],
        out_shape=[
            jax.ShapeDtypeStruct((2, n), jnp.float32),
            jax.ShapeDtypeStruct((2, n), jnp.int32),
            jax.ShapeDtypeStruct((rows, 2), jnp.float32),
            jax.ShapeDtypeStruct((rows, 2), jnp.int32),
        ],
    )(y)


def _z_body(z_ref, i3_ref, i4_ref, z1_ref, i5_ref, v_s, p_s):
    g = pl.program_id(0)
    w = z_ref.shape[2]
    blk = z_ref[...]

    _, js = _topk_axis(blk, 2, 0, largest=True)
    i3_ref[...] = jnp.concatenate(js, axis=0)

    _, js = _topk_axis(blk, 3, 1, largest=True)
    i4_ref[...] = jnp.concatenate(js, axis=1)

    lanes = jax.lax.broadcasted_iota(jnp.int32, blk.shape, 2) + g * w
    lm = jnp.max(blk, axis=2)
    lj = jnp.min(jnp.where(blk == lm[:, :, None], lanes, _BIG_I32), axis=2)

    @pl.when(g == 0)
    def _():
        v_s[...] = lm
        p_s[...] = lj

    @pl.when(g > 0)
    def _():
        take = lm > v_s[...]
        v_s[...] = jnp.where(take, lm, v_s[...])
        p_s[...] = jnp.where(take, lj, p_s[...])

    @pl.when(g == pl.num_programs(0) - 1)
    def _():
        z1_ref[...] = v_s[...]
        i5_ref[...] = p_s[...]


def _topk_z(z):
    a0, a1, a2 = z.shape
    w = 1024
    return pl.pallas_call(
        _z_body,
        grid=(a2 // w,),
        in_specs=[pl.BlockSpec((a0, a1, w), lambda g: (0, 0, g))],
        out_specs=[
            pl.BlockSpec((2, a1, w), lambda g: (0, 0, g)),
            pl.BlockSpec((a0, 3, w), lambda g: (0, 0, g)),
            pl.BlockSpec((a0, a1), lambda g: (0, 0)),
            pl.BlockSpec((a0, a1), lambda g: (0, 0)),
        ],
        out_shape=[
            jax.ShapeDtypeStruct((2, a1, a2), jnp.int32),
            jax.ShapeDtypeStruct((a0, 3, a2), jnp.int32),
            jax.ShapeDtypeStruct((a0, a1), jnp.float32),
            jax.ShapeDtypeStruct((a0, a1), jnp.int32),
        ],
        scratch_shapes=[
            pltpu.VMEM((a0, a1), jnp.float32),
            pltpu.VMEM((a0, a1), jnp.int32),
        ],
    )(z)


def _d_body(d_ref, d2_ref, i8_ref, d3_ref, i9_ref):
    blk = d_ref[...]

    vs, js = _topk_axis(blk, 2, 1, largest=True)
    d2_ref[...] = jnp.concatenate(vs, axis=1)
    i8_ref[...] = jnp.concatenate(js, axis=1)

    vs, js = _topk_axis(blk, 2, 2, largest=True)
    d3_ref[...] = jnp.concatenate(vs, axis=2)
    i9_ref[...] = jnp.concatenate(js, axis=2)


def _topk_d(dr):
    b, a1, a2 = dr.shape
    bb = 4
    return pl.pallas_call(
        _d_body,
        grid=(b // bb,),
        in_specs=[pl.BlockSpec((bb, a1, a2), lambda g: (g, 0, 0))],
        out_specs=[
            pl.BlockSpec((bb, 2, a2), lambda g: (g, 0, 0)),
            pl.BlockSpec((bb, 2, a2), lambda g: (g, 0, 0)),
            pl.BlockSpec((bb, a1, 2), lambda g: (g, 0, 0)),
            pl.BlockSpec((bb, a1, 2), lambda g: (g, 0, 0)),
        ],
        out_shape=[
            jax.ShapeDtypeStruct((b, 2, a2), jnp.float32),
            jax.ShapeDtypeStruct((b, 2, a2), jnp.int32),
            jax.ShapeDtypeStruct((b, a1, 2), jnp.float32),
            jax.ShapeDtypeStruct((b, a1, 2), jnp.int32),
        ],
    )(dr)


def kernel(x, y, z, d):
    x0, i0 = _topk_x(x)
    y1, i1, y2, i2 = _topk_y(y)
    i3, i4, z1, i5 = _topk_z(z)
    z1 = z1[:, :, None]
    i5 = i5[:, :, None]

    b0, b1, a1, a2 = d.shape
    d2, i8, d3, i9 = _topk_d(d.reshape(b0 * b1, a1, a2))
    d2 = d2.reshape(b0, b1, 2, a2)
    i8 = i8.reshape(b0, b1, 2, a2)
    d3 = d3.reshape(b0, b1, a1, 2)
    i9 = i9.reshape(b0, b1, a1, 2)

    return (x0, y1, y2, i0, i1, i2, z1, i3, i4, i5, d2, d3, i8, i9)


# speedup vs baseline: 28.9377x; 1.0230x over previous
import jax
import jax.numpy as jnp
from jax.experimental import pallas as pl
from jax.experimental.pallas import tpu as pltpu

_BIG_I32 = 2**30
_NEG_INF = float("-inf")
_POS_INF = float("inf")


def _topk_axis(blk, k, axis, largest=True, idx_base=0):
    idx = jax.lax.broadcasted_iota(jnp.int32, blk.shape, axis) + idx_base
    fill = _NEG_INF if largest else _POS_INF
    cur = blk
    vs, js = [], []
    for t in range(k):
        if largest:
            m = jnp.max(cur, axis=axis, keepdims=True)
        else:
            m = jnp.min(cur, axis=axis, keepdims=True)
        j = jnp.min(jnp.where(cur == m, idx, _BIG_I32), axis=axis, keepdims=True)
        vs.append(m)
        js.append(j)
        if t + 1 < k:
            cur = jnp.where(idx == j, fill, cur)
    return vs, js


def _x_body(x_ref, v_ref, i_ref):
    vs, js = _topk_axis(x_ref[...], 4, 1, largest=True)
    v_ref[...] = jnp.concatenate(vs, axis=1)
    i_ref[...] = jnp.concatenate(js, axis=1)


def _topk_x(x):
    rows, n = x.shape
    blk_r = 32
    return pl.pallas_call(
        _x_body,
        grid=(rows // blk_r,),
        in_specs=[pl.BlockSpec((blk_r, n), lambda g: (g, 0))],
        out_specs=[
            pl.BlockSpec((blk_r, 4), lambda g: (g, 0)),
            pl.BlockSpec((blk_r, 4), lambda g: (g, 0)),
        ],
        out_shape=[
            jax.ShapeDtypeStruct((rows, 4), jnp.float32),
            jax.ShapeDtypeStruct((rows, 4), jnp.int32),
        ],
    )(x)


def _y_body(y_ref, y1_ref, i1_ref, y2_ref, i2_ref):
    g = pl.program_id(0)
    blk = y_ref[...]
    blk_r = blk.shape[0]

    (m1, m2), (j1, j2) = _topk_axis(blk, 2, 0, largest=True, idx_base=g * blk_r)

    @pl.when(g == 0)
    def _():
        y1_ref[...] = jnp.concatenate([m1, m2], axis=0)
        i1_ref[...] = jnp.concatenate([j1, j2], axis=0)

    @pl.when(g > 0)
    def _():
        v1 = y1_ref[0:1, :]
        v2 = y1_ref[1:2, :]
        p1 = i1_ref[0:1, :]
        p2 = i1_ref[1:2, :]
        take1 = m1 > v1
        a_v = jnp.where(take1, v1, v2)
        a_i = jnp.where(take1, p1, p2)
        b_v = jnp.where(take1, m2, m1)
        b_i = jnp.where(take1, j2, j1)
        take2 = b_v > a_v
        y1_ref[...] = jnp.concatenate(
            [jnp.where(take1, m1, v1), jnp.where(take2, b_v, a_v)], axis=0)
        i1_ref[...] = jnp.concatenate(
            [jnp.where(take1, j1, p1), jnp.where(take2, b_i, a_i)], axis=0)

    (n1, n2), (k1, k2) = _topk_axis(blk, 2, 1, largest=False)
    y2_ref[...] = jnp.concatenate([n1, n2], axis=1)
    i2_ref[...] = jnp.concatenate([k1, k2], axis=1)


def _topk_y(y):
    rows, n = y.shape
    blk_r = 256
    grid = rows // blk_r
    return pl.pallas_call(
        _y_body,
        grid=(grid,),
        in_specs=[pl.BlockSpec((blk_r, n), lambda g: (g, 0))],
        out_specs=[
            pl.BlockSpec((2, n), lambda g: (0, 0)),
            pl.BlockSpec((2, n), lambda g: (0, 0)),
            pl.BlockSpec((blk_r, 2), lambda g: (g, 0)),
            pl.BlockSpec((blk_r, 2), lambda g: (g, 0)),
        ],
        out_shape=[
            jax.ShapeDtypeStruct((2, n), jnp.float32),
            jax.ShapeDtypeStruct((2, n), jnp.int32),
            jax.ShapeDtypeStruct((rows, 2), jnp.float32),
            jax.ShapeDtypeStruct((rows, 2), jnp.int32),
        ],
    )(y)


def _z_body(z_ref, i3_ref, i4_ref, z1_ref, i5_ref, v_s, p_s):
    g = pl.program_id(0)
    w = z_ref.shape[2]
    blk = z_ref[...]

    _, js = _topk_axis(blk, 2, 0, largest=True)
    i3_ref[...] = jnp.concatenate(js, axis=0)

    _, js = _topk_axis(blk, 3, 1, largest=True)
    i4_ref[...] = jnp.concatenate(js, axis=1)

    lanes = jax.lax.broadcasted_iota(jnp.int32, blk.shape, 2) + g * w
    lm = jnp.max(blk, axis=2)
    lj = jnp.min(jnp.where(blk == lm[:, :, None], lanes, _BIG_I32), axis=2)

    @pl.when(g == 0)
    def _():
        v_s[...] = lm
        p_s[...] = lj

    @pl.when(g > 0)
    def _():
        take = lm > v_s[...]
        v_s[...] = jnp.where(take, lm, v_s[...])
        p_s[...] = jnp.where(take, lj, p_s[...])

    @pl.when(g == pl.num_programs(0) - 1)
    def _():
        z1_ref[...] = v_s[...]
        i5_ref[...] = p_s[...]


def _topk_z(z):
    a0, a1, a2 = z.shape
    w = 512
    return pl.pallas_call(
        _z_body,
        grid=(a2 // w,),
        in_specs=[pl.BlockSpec((a0, a1, w), lambda g: (0, 0, g))],
        out_specs=[
            pl.BlockSpec((2, a1, w), lambda g: (0, 0, g)),
            pl.BlockSpec((a0, 3, w), lambda g: (0, 0, g)),
            pl.BlockSpec((a0, a1), lambda g: (0, 0)),
            pl.BlockSpec((a0, a1), lambda g: (0, 0)),
        ],
        out_shape=[
            jax.ShapeDtypeStruct((2, a1, a2), jnp.int32),
            jax.ShapeDtypeStruct((a0, 3, a2), jnp.int32),
            jax.ShapeDtypeStruct((a0, a1), jnp.float32),
            jax.ShapeDtypeStruct((a0, a1), jnp.int32),
        ],
        scratch_shapes=[
            pltpu.VMEM((a0, a1), jnp.float32),
            pltpu.VMEM((a0, a1), jnp.int32),
        ],
    )(z)


def _d_body(d_ref, d2_ref, i8_ref, d3_ref, i9_ref):
    blk = d_ref[...]

    vs, js = _topk_axis(blk, 2, 1, largest=True)
    d2_ref[...] = jnp.concatenate(vs, axis=1)
    i8_ref[...] = jnp.concatenate(js, axis=1)

    vs, js = _topk_axis(blk, 2, 2, largest=True)
    d3_ref[...] = jnp.concatenate(vs, axis=2)
    i9_ref[...] = jnp.concatenate(js, axis=2)


def _topk_d(dr):
    b, a1, a2 = dr.shape
    bb = 8
    return pl.pallas_call(
        _d_body,
        grid=(b // bb,),
        in_specs=[pl.BlockSpec((bb, a1, a2), lambda g: (g, 0, 0))],
        out_specs=[
            pl.BlockSpec((bb, 2, a2), lambda g: (g, 0, 0)),
            pl.BlockSpec((bb, 2, a2), lambda g: (g, 0, 0)),
            pl.BlockSpec((bb, a1, 2), lambda g: (g, 0, 0)),
            pl.BlockSpec((bb, a1, 2), lambda g: (g, 0, 0)),
        ],
        out_shape=[
            jax.ShapeDtypeStruct((b, 2, a2), jnp.float32),
            jax.ShapeDtypeStruct((b, 2, a2), jnp.int32),
            jax.ShapeDtypeStruct((b, a1, 2), jnp.float32),
            jax.ShapeDtypeStruct((b, a1, 2), jnp.int32),
        ],
    )(dr)


def kernel(x, y, z, d):
    x0, i0 = _topk_x(x)
    y1, i1, y2, i2 = _topk_y(y)
    i3, i4, z1, i5 = _topk_z(z)
    z1 = z1[:, :, None]
    i5 = i5[:, :, None]

    b0, b1, a1, a2 = d.shape
    d2, i8, d3, i9 = _topk_d(d.reshape(b0 * b1, a1, a2))
    d2 = d2.reshape(b0, b1, 2, a2)
    i8 = i8.reshape(b0, b1, 2, a2)
    d3 = d3.reshape(b0, b1, a1, 2)
    i9 = i9.reshape(b0, b1, a1, 2)

    return (x0, y1, y2, i0, i1, i2, z1, i3, i4, i5, d2, d3, i8, i9)


# speedup vs baseline: 29.0821x; 1.0050x over previous
import jax
import jax.numpy as jnp
from jax.experimental import pallas as pl
from jax.experimental.pallas import tpu as pltpu

_BIG_I32 = 2**30
_NEG_INF = float("-inf")
_POS_INF = float("inf")

_XB, _YB, _ZW, _DB = 32, 256, 512, 8
_XS, _YS, _ZS, _DS = 4, 16, 4, 16
_Y0, _Z0, _D0 = _XS, _XS + _YS, _XS + _YS + _ZS
_STEPS = _XS + _YS + _ZS + _DS


def _topk_axis(blk, k, axis, largest=True, idx_base=0):
    idx = jax.lax.broadcasted_iota(jnp.int32, blk.shape, axis) + idx_base
    fill = _NEG_INF if largest else _POS_INF
    cur = blk
    vs, js = [], []
    for t in range(k):
        if largest:
            m = jnp.max(cur, axis=axis, keepdims=True)
        else:
            m = jnp.min(cur, axis=axis, keepdims=True)
        j = jnp.min(jnp.where(cur == m, idx, _BIG_I32), axis=axis, keepdims=True)
        vs.append(m)
        js.append(j)
        if t + 1 < k:
            cur = jnp.where(idx == j, fill, cur)
    return vs, js


def _body(x_ref, y_ref, z_ref, d_ref,
          xv_ref, xi_ref, y1_ref, i1_ref, y2_ref, i2_ref,
          i3_ref, i4_ref, z1_ref, i5_ref,
          d2_ref, i8_ref, d3_ref, i9_ref,
          v_s, p_s):
    g = pl.program_id(0)

    @pl.when(g < _Y0)
    def _x_phase():
        vs, js = _topk_axis(x_ref[...], 4, 1, largest=True)
        xv_ref[...] = jnp.concatenate(vs, axis=1)
        xi_ref[...] = jnp.concatenate(js, axis=1)

    @pl.when(jnp.logical_and(g >= _Y0, g < _Z0))
    def _y_phase():
        gy = g - _Y0
        blk = y_ref[...]

        (m1, m2), (j1, j2) = _topk_axis(blk, 2, 0, largest=True,
                                        idx_base=gy * _YB)

        @pl.when(gy == 0)
        def _():
            y1_ref[...] = jnp.concatenate([m1, m2], axis=0)
            i1_ref[...] = jnp.concatenate([j1, j2], axis=0)

        @pl.when(gy > 0)
        def _():
            v1 = y1_ref[0:1, :]
            v2 = y1_ref[1:2, :]
            p1 = i1_ref[0:1, :]
            p2 = i1_ref[1:2, :]
            take1 = m1 > v1
            a_v = jnp.where(take1, v1, v2)
            a_i = jnp.where(take1, p1, p2)
            b_v = jnp.where(take1, m2, m1)
            b_i = jnp.where(take1, j2, j1)
            take2 = b_v > a_v
            y1_ref[...] = jnp.concatenate(
                [jnp.where(take1, m1, v1), jnp.where(take2, b_v, a_v)], axis=0)
            i1_ref[...] = jnp.concatenate(
                [jnp.where(take1, j1, p1), jnp.where(take2, b_i, a_i)], axis=0)

        (n1, n2), (k1, k2) = _topk_axis(blk, 2, 1, largest=False)
        y2_ref[...] = jnp.concatenate([n1, n2], axis=1)
        i2_ref[...] = jnp.concatenate([k1, k2], axis=1)

    @pl.when(jnp.logical_and(g >= _Z0, g < _D0))
    def _z_phase():
        gz = g - _Z0
        blk = z_ref[...]

        _, js = _topk_axis(blk, 2, 0, largest=True)
        i3_ref[...] = jnp.concatenate(js, axis=0)

        _, js = _topk_axis(blk, 3, 1, largest=True)
        i4_ref[...] = jnp.concatenate(js, axis=1)

        lanes = jax.lax.broadcasted_iota(jnp.int32, blk.shape, 2) + gz * _ZW
        lm = jnp.max(blk, axis=2)
        lj = jnp.min(jnp.where(blk == lm[:, :, None], lanes, _BIG_I32), axis=2)

        @pl.when(gz == 0)
        def _():
            v_s[...] = lm
            p_s[...] = lj

        @pl.when(gz > 0)
        def _():
            take = lm > v_s[...]
            v_s[...] = jnp.where(take, lm, v_s[...])
            p_s[...] = jnp.where(take, lj, p_s[...])

        @pl.when(gz == _ZS - 1)
        def _():
            z1_ref[...] = v_s[...]
            i5_ref[...] = p_s[...]

    @pl.when(g >= _D0)
    def _d_phase():
        blk = d_ref[...]

        vs, js = _topk_axis(blk, 2, 1, largest=True)
        d2_ref[...] = jnp.concatenate(vs, axis=1)
        i8_ref[...] = jnp.concatenate(js, axis=1)

        vs, js = _topk_axis(blk, 2, 2, largest=True)
        d3_ref[...] = jnp.concatenate(vs, axis=2)
        i9_ref[...] = jnp.concatenate(js, axis=2)


def _fused(x, y, z, d):
    xr, xn = x.shape
    yr, yn = y.shape
    a0, a1, a2 = z.shape
    db, d1, d2n = d.shape

    def xm(g):
        return (jnp.clip(g, 0, _XS - 1), 0)

    def ym(g):
        return (jnp.clip(g - _Y0, 0, _YS - 1), 0)

    def zm(g):
        return (0, 0, jnp.clip(g - _Z0, 0, _ZS - 1))

    def dm(g):
        return (jnp.clip(g - _D0, 0, _DS - 1), 0, 0)

    return pl.pallas_call(
        _body,
        grid=(_STEPS,),
        in_specs=[
            pl.BlockSpec((_XB, xn), xm),
            pl.BlockSpec((_YB, yn), ym),
            pl.BlockSpec((a0, a1, _ZW), zm),
            pl.BlockSpec((_DB, d1, d2n), dm),
        ],
        out_specs=[
            pl.BlockSpec((_XB, 4), xm),
            pl.BlockSpec((_XB, 4), xm),
            pl.BlockSpec((2, yn), lambda g: (0, 0)),
            pl.BlockSpec((2, yn), lambda g: (0, 0)),
            pl.BlockSpec((_YB, 2), ym),
            pl.BlockSpec((_YB, 2), ym),
            pl.BlockSpec((2, a1, _ZW), zm),
            pl.BlockSpec((a0, 3, _ZW), zm),
            pl.BlockSpec((a0, a1), lambda g: (0, 0)),
            pl.BlockSpec((a0, a1), lambda g: (0, 0)),
            pl.BlockSpec((_DB, 2, d2n), dm),
            pl.BlockSpec((_DB, 2, d2n), dm),
            pl.BlockSpec((_DB, d1, 2), dm),
            pl.BlockSpec((_DB, d1, 2), dm),
        ],
        out_shape=[
            jax.ShapeDtypeStruct((xr, 4), jnp.float32),
            jax.ShapeDtypeStruct((xr, 4), jnp.int32),
            jax.ShapeDtypeStruct((2, yn), jnp.float32),
            jax.ShapeDtypeStruct((2, yn), jnp.int32),
            jax.ShapeDtypeStruct((yr, 2), jnp.float32),
            jax.ShapeDtypeStruct((yr, 2), jnp.int32),
            jax.ShapeDtypeStruct((2, a1, a2), jnp.int32),
            jax.ShapeDtypeStruct((a0, 3, a2), jnp.int32),
            jax.ShapeDtypeStruct((a0, a1), jnp.float32),
            jax.ShapeDtypeStruct((a0, a1), jnp.int32),
            jax.ShapeDtypeStruct((db, 2, d2n), jnp.float32),
            jax.ShapeDtypeStruct((db, 2, d2n), jnp.int32),
            jax.ShapeDtypeStruct((db, d1, 2), jnp.float32),
            jax.ShapeDtypeStruct((db, d1, 2), jnp.int32),
        ],
        scratch_shapes=[
            pltpu.VMEM((a0, a1), jnp.float32),
            pltpu.VMEM((a0, a1), jnp.int32),
        ],
    )(x, y, z, d)


def kernel(x, y, z, d):
    b0, b1, a1, a2 = d.shape
    (x0, i0, y1, i1, y2, i2, i3, i4, z1, i5,
     d2, i8, d3, i9) = _fused(x, y, z, d.reshape(b0 * b1, a1, a2))
    z1 = z1[:, :, None]
    i5 = i5[:, :, None]
    d2 = d2.reshape(b0, b1, 2, a2)
    i8 = i8.reshape(b0, b1, 2, a2)
    d3 = d3.reshape(b0, b1, a1, 2)
    i9 = i9.reshape(b0, b1, a1, 2)
    return (x0, y1, y2, i0, i1, i2, z1, i3, i4, i5, d2, d3, i8, i9)


# speedup vs baseline: 29.9561x; 1.0301x over previous
import functools

import jax
import jax.numpy as jnp
from jax import lax
from jax.experimental import pallas as pl
from jax.experimental.pallas import tpu as pltpu
from jax.experimental.pallas import tpu_sc as plsc

_BIG_I32 = 2**30
_NEG_INF = float("-inf")
_POS_INF = float("inf")


_XN = 32768
_XROWS_W = 4


def _x_sc_body(x_hbm, xv_hbm, xi_hbm, row_v, outv_v, outi_v):
    wid = lax.axis_index("c") * 16 + lax.axis_index("s")
    lane = lax.broadcasted_iota(jnp.int32, (16,), 0)
    neg = jnp.full((16,), _NEG_INF, jnp.float32)
    zero = jnp.zeros((16,), jnp.int32)

    for rr in range(_XROWS_W):
        row = wid * _XROWS_W + rr
        pltpu.sync_copy(x_hbm.at[row], row_v)

        def step(i, st):
            v1, v2, v3, v4, t1, t2, t3, t4 = st
            cur = row_v[pl.ds(i * 16, 16)]
            ti = jnp.full((16,), i, jnp.int32)
            gt1 = cur > v1
            gt2 = cur > v2
            gt3 = cur > v3
            gt4 = cur > v4
            v4 = jnp.where(gt3, v3, jnp.where(gt4, cur, v4))
            t4 = jnp.where(gt3, t3, jnp.where(gt4, ti, t4))
            v3 = jnp.where(gt2, v2, jnp.where(gt3, cur, v3))
            t3 = jnp.where(gt2, t2, jnp.where(gt3, ti, t3))
            v2 = jnp.where(gt1, v1, jnp.where(gt2, cur, v2))
            t2 = jnp.where(gt1, t1, jnp.where(gt2, ti, t2))
            v1 = jnp.where(gt1, cur, v1)
            t1 = jnp.where(gt1, ti, t1)
            return (v1, v2, v3, v4, t1, t2, t3, t4)

        v1, v2, v3, v4, t1, t2, t3, t4 = lax.fori_loop(
            0, _XN // 16, step,
            (neg, neg, neg, neg, zero, zero, zero, zero), unroll=8)

        negi = jnp.full((16,), -_BIG_I32, jnp.int32)

        def _allmax(v):
            for s in (8, 4, 2, 1):
                perm = jnp.bitwise_xor(lane, s)
                v = jnp.maximum(v, v.at[perm].get(mode="promise_in_bounds"))
            return v

        accv = jnp.zeros((16,), jnp.float32)
        acci = jnp.zeros((16,), jnp.int32)
        for r in range(4):
            m = _allmax(v1)
            gidx = t1 * 16 + lane
            j = -_allmax(jnp.where(v1 == m, -gidx, negi))
            accv = jnp.where(lane == r, m, accv)
            acci = jnp.where(lane == r, j, acci)
            if r < 3:
                msk = jnp.logical_and(v1 == m, gidx == j)
                v1 = jnp.where(msk, v2, v1)
                t1 = jnp.where(msk, t2, t1)
                v2 = jnp.where(msk, v3, v2)
                t2 = jnp.where(msk, t3, t2)
                v3 = jnp.where(msk, v4, v3)
                t3 = jnp.where(msk, t4, t3)
                v4 = jnp.where(msk, neg, v4)
        outv_v[rr] = accv
        outi_v[rr] = acci

    base = wid * _XROWS_W
    pltpu.sync_copy(outv_v, xv_hbm.at[pl.ds(base, _XROWS_W)])
    pltpu.sync_copy(outi_v, xi_hbm.at[pl.ds(base, _XROWS_W)])


def _topk_x_sc(x):
    rows = x.shape[0]
    mesh = plsc.VectorSubcoreMesh(core_axis_name="c", subcore_axis_name="s")
    kern = pl.kernel(
        _x_sc_body,
        out_type=[
            jax.ShapeDtypeStruct((rows, 16), jnp.float32),
            jax.ShapeDtypeStruct((rows, 16), jnp.int32),
        ],
        mesh=mesh,
        scratch_types=[
            pltpu.VMEM((_XN,), jnp.float32),
            pltpu.VMEM((_XROWS_W, 16), jnp.float32),
            pltpu.VMEM((_XROWS_W, 16), jnp.int32),
        ],
    )
    xv, xi = kern(x)
    return xv[:, :4], xi[:, :4]


_YB, _ZW, _DB = 256, 512, 8
_YS, _ZS, _DS = 16, 4, 16
_Z0, _D0 = _YS, _YS + _ZS
_STEPS = _YS + _ZS + _DS


def _topk_axis(blk, k, axis, largest=True, idx_base=0):
    idx = jax.lax.broadcasted_iota(jnp.int32, blk.shape, axis) + idx_base
    fill = _NEG_INF if largest else _POS_INF
    cur = blk
    vs, js = [], []
    for t in range(k):
        if largest:
            m = jnp.max(cur, axis=axis, keepdims=True)
        else:
            m = jnp.min(cur, axis=axis, keepdims=True)
        j = jnp.min(jnp.where(cur == m, idx, _BIG_I32), axis=axis, keepdims=True)
        vs.append(m)
        js.append(j)
        if t + 1 < k:
            cur = jnp.where(idx == j, fill, cur)
    return vs, js


def _body(y_ref, z_ref, d_ref,
          y1_ref, i1_ref, y2_ref, i2_ref,
          i3_ref, i4_ref, z1_ref, i5_ref,
          d2_ref, i8_ref, d3_ref, i9_ref,
          v_s, p_s):
    g = pl.program_id(0)

    @pl.when(g < _Z0)
    def _y_phase():
        gy = g
        blk = y_ref[...]

        (m1, m2), (j1, j2) = _topk_axis(blk, 2, 0, largest=True,
                                        idx_base=gy * _YB)

        @pl.when(gy == 0)
        def _():
            y1_ref[...] = jnp.concatenate([m1, m2], axis=0)
            i1_ref[...] = jnp.concatenate([j1, j2], axis=0)

        @pl.when(gy > 0)
        def _():
            v1 = y1_ref[0:1, :]
            v2 = y1_ref[1:2, :]
            p1 = i1_ref[0:1, :]
            p2 = i1_ref[1:2, :]
            take1 = m1 > v1
            a_v = jnp.where(take1, v1, v2)
            a_i = jnp.where(take1, p1, p2)
            b_v = jnp.where(take1, m2, m1)
            b_i = jnp.where(take1, j2, j1)
            take2 = b_v > a_v
            y1_ref[...] = jnp.concatenate(
                [jnp.where(take1, m1, v1), jnp.where(take2, b_v, a_v)], axis=0)
            i1_ref[...] = jnp.concatenate(
                [jnp.where(take1, j1, p1), jnp.where(take2, b_i, a_i)], axis=0)

        (n1, n2), (k1, k2) = _topk_axis(blk, 2, 1, largest=False)
        y2_ref[...] = jnp.concatenate([n1, n2], axis=1)
        i2_ref[...] = jnp.concatenate([k1, k2], axis=1)

    @pl.when(jnp.logical_and(g >= _Z0, g < _D0))
    def _z_phase():
        gz = g - _Z0
        blk = z_ref[...]

        _, js = _topk_axis(blk, 2, 0, largest=True)
        i3_ref[...] = jnp.concatenate(js, axis=0)

        _, js = _topk_axis(blk, 3, 1, largest=True)
        i4_ref[...] = jnp.concatenate(js, axis=1)

        lanes = jax.lax.broadcasted_iota(jnp.int32, blk.shape, 2) + gz * _ZW
        lm = jnp.max(blk, axis=2)
        lj = jnp.min(jnp.where(blk == lm[:, :, None], lanes, _BIG_I32), axis=2)

        @pl.when(gz == 0)
        def _():
            v_s[...] = lm
            p_s[...] = lj

        @pl.when(gz > 0)
        def _():
            take = lm > v_s[...]
            v_s[...] = jnp.where(take, lm, v_s[...])
            p_s[...] = jnp.where(take, lj, p_s[...])

        @pl.when(gz == _ZS - 1)
        def _():
            z1_ref[...] = v_s[...]
            i5_ref[...] = p_s[...]

    @pl.when(g >= _D0)
    def _d_phase():
        blk = d_ref[...]

        vs, js = _topk_axis(blk, 2, 1, largest=True)
        d2_ref[...] = jnp.concatenate(vs, axis=1)
        i8_ref[...] = jnp.concatenate(js, axis=1)

        vs, js = _topk_axis(blk, 2, 2, largest=True)
        d3_ref[...] = jnp.concatenate(vs, axis=2)
        i9_ref[...] = jnp.concatenate(js, axis=2)


def _fused(y, z, d):
    yr, yn = y.shape
    a0, a1, a2 = z.shape
    db, d1, d2n = d.shape

    def ym(g):
        return (jnp.clip(g, 0, _YS - 1), 0)

    def zm(g):
        return (0, 0, jnp.clip(g - _Z0, 0, _ZS - 1))

    def dm(g):
        return (jnp.clip(g - _D0, 0, _DS - 1), 0, 0)

    return pl.pallas_call(
        _body,
        grid=(_STEPS,),
        in_specs=[
            pl.BlockSpec((_YB, yn), ym),
            pl.BlockSpec((a0, a1, _ZW), zm),
            pl.BlockSpec((_DB, d1, d2n), dm),
        ],
        out_specs=[
            pl.BlockSpec((2, yn), lambda g: (0, 0)),
            pl.BlockSpec((2, yn), lambda g: (0, 0)),
            pl.BlockSpec((_YB, 2), ym),
            pl.BlockSpec((_YB, 2), ym),
            pl.BlockSpec((2, a1, _ZW), zm),
            pl.BlockSpec((a0, 3, _ZW), zm),
            pl.BlockSpec((a0, a1), lambda g: (0, 0)),
            pl.BlockSpec((a0, a1), lambda g: (0, 0)),
            pl.BlockSpec((_DB, 2, d2n), dm),
            pl.BlockSpec((_DB, 2, d2n), dm),
            pl.BlockSpec((_DB, d1, 2), dm),
            pl.BlockSpec((_DB, d1, 2), dm),
        ],
        out_shape=[
            jax.ShapeDtypeStruct((2, yn), jnp.float32),
            jax.ShapeDtypeStruct((2, yn), jnp.int32),
            jax.ShapeDtypeStruct((yr, 2), jnp.float32),
            jax.ShapeDtypeStruct((yr, 2), jnp.int32),
            jax.ShapeDtypeStruct((2, a1, a2), jnp.int32),
            jax.ShapeDtypeStruct((a0, 3, a2), jnp.int32),
            jax.ShapeDtypeStruct((a0, a1), jnp.float32),
            jax.ShapeDtypeStruct((a0, a1), jnp.int32),
            jax.ShapeDtypeStruct((db, 2, d2n), jnp.float32),
            jax.ShapeDtypeStruct((db, 2, d2n), jnp.int32),
            jax.ShapeDtypeStruct((db, d1, 2), jnp.float32),
            jax.ShapeDtypeStruct((db, d1, 2), jnp.int32),
        ],
        scratch_shapes=[
            pltpu.VMEM((a0, a1), jnp.float32),
            pltpu.VMEM((a0, a1), jnp.int32),
        ],
    )(y, z, d)


def kernel(x, y, z, d):
    x0, i0 = _topk_x_sc(x)

    b0, b1, a1, a2 = d.shape
    (y1, i1, y2, i2, i3, i4, z1, i5,
     d2, i8, d3, i9) = _fused(y, z, d.reshape(b0 * b1, a1, a2))
    z1 = z1[:, :, None]
    i5 = i5[:, :, None]
    d2 = d2.reshape(b0, b1, 2, a2)
    i8 = i8.reshape(b0, b1, 2, a2)
    d3 = d3.reshape(b0, b1, a1, 2)
    i9 = i9.reshape(b0, b1, a1, 2)
    return (x0, y1, y2, i0, i1, i2, z1, i3, i4, i5, d2, d3, i8, i9)


# speedup vs baseline: 34.4624x; 1.1504x over previous
import functools

import jax
import jax.numpy as jnp
from jax import lax
from jax.experimental import pallas as pl
from jax.experimental.pallas import tpu as pltpu
from jax.experimental.pallas import tpu_sc as plsc

_BIG_I32 = 2**30
_NEG_INF = float("-inf")
_POS_INF = float("inf")


_XN = 32768
_XROWS_W = 4
_DN = 1024
_DROWS = 16384
_DROWS_W = _DROWS // 32
_DCH = 32
_DNCH = _DROWS_W // _DCH


def _lane_iota():
    return lax.broadcasted_iota(jnp.int32, (16,), 0)


def _allmax(v):
    lane = _lane_iota()
    for s in (8, 4, 2, 1):
        perm = jnp.bitwise_xor(lane, s)
        v = jnp.maximum(v, v.at[perm].get(mode="promise_in_bounds"))
    return v


def _merge_rounds(vs, ts, k, accv, acci):
    lane = _lane_iota()
    neg = jnp.full((16,), _NEG_INF, jnp.float32)
    negi = jnp.full((16,), -_BIG_I32, jnp.int32)
    vs = list(vs)
    ts = list(ts)
    for r in range(k):
        m = _allmax(vs[0])
        gidx = ts[0] * 16 + lane
        j = -_allmax(jnp.where(vs[0] == m, -gidx, negi))
        accv = jnp.where(lane == r, m, accv)
        acci = jnp.where(lane == r, j, acci)
        if r + 1 < k:
            msk = jnp.logical_and(vs[0] == m, gidx == j)
            for q in range(len(vs) - 1):
                vs[q] = jnp.where(msk, vs[q + 1], vs[q])
                ts[q] = jnp.where(msk, ts[q + 1], ts[q])
            vs[-1] = jnp.where(msk, neg, vs[-1])
    return accv, acci


def _sc_body(x_hbm, d_hbm, xv_hbm, xi_hbm, dv_hbm, di_hbm,
             row_v, db0_v, db1_v, xo_v, xo_i, do_v, do_i,
             sem0, sem1):
    wid = lax.axis_index("c") * 16 + lax.axis_index("s")
    neg = jnp.full((16,), _NEG_INF, jnp.float32)
    zero = jnp.zeros((16,), jnp.int32)

    for rr in range(_XROWS_W):
        row = wid * _XROWS_W + rr
        pltpu.sync_copy(x_hbm.at[row], row_v)

        def xstep(i, st):
            v1, v2, v3, v4, t1, t2, t3, t4 = st
            cur = row_v[pl.ds(i * 16, 16)]
            ti = jnp.full((16,), i, jnp.int32)
            gt1 = cur > v1
            gt2 = cur > v2
            gt3 = cur > v3
            gt4 = cur > v4
            v4 = jnp.where(gt3, v3, jnp.where(gt4, cur, v4))
            t4 = jnp.where(gt3, t3, jnp.where(gt4, ti, t4))
            v3 = jnp.where(gt2, v2, jnp.where(gt3, cur, v3))
            t3 = jnp.where(gt2, t2, jnp.where(gt3, ti, t3))
            v2 = jnp.where(gt1, v1, jnp.where(gt2, cur, v2))
            t2 = jnp.where(gt1, t1, jnp.where(gt2, ti, t2))
            v1 = jnp.where(gt1, cur, v1)
            t1 = jnp.where(gt1, ti, t1)
            return (v1, v2, v3, v4, t1, t2, t3, t4)

        v1, v2, v3, v4, t1, t2, t3, t4 = lax.fori_loop(
            0, _XN // 16, xstep,
            (neg, neg, neg, neg, zero, zero, zero, zero), unroll=8)

        accv, acci = _merge_rounds(
            (v1, v2, v3, v4), (t1, t2, t3, t4), 4,
            jnp.zeros((16,), jnp.float32), jnp.zeros((16,), jnp.int32))
        xo_v[rr] = accv
        xo_i[rr] = acci

    xbase = wid * _XROWS_W
    pltpu.sync_copy(xo_v, xv_hbm.at[pl.ds(xbase, _XROWS_W)])
    pltpu.sync_copy(xo_i, xi_hbm.at[pl.ds(xbase, _XROWS_W)])

    dbase = wid * _DROWS_W
    bufs = (db0_v, db1_v)
    sems = (sem0, sem1)
    copies = [None, None]
    copies[0] = pltpu.async_copy(
        d_hbm.at[pl.ds(dbase, _DCH)], bufs[0], sems[0])
    for c in range(_DNCH):
        buf = bufs[c % 2]
        if c + 1 < _DNCH:
            copies[(c + 1) % 2] = pltpu.async_copy(
                d_hbm.at[pl.ds(dbase + (c + 1) * _DCH, _DCH)],
                bufs[(c + 1) % 2], sems[(c + 1) % 2])
        copies[c % 2].wait()

        def rowbody(rr, _):
            def dstep(i, st):
                v1, v2, t1, t2 = st
                cur = buf[rr, pl.ds(i * 16, 16)]
                ti = jnp.full((16,), i, jnp.int32)
                gt1 = cur > v1
                gt2 = cur > v2
                v2n = jnp.where(gt1, v1, jnp.where(gt2, cur, v2))
                t2n = jnp.where(gt1, t1, jnp.where(gt2, ti, t2))
                v1n = jnp.where(gt1, cur, v1)
                t1n = jnp.where(gt1, ti, t1)
                return (v1n, v2n, t1n, t2n)

            v1, v2, t1, t2 = lax.fori_loop(
                0, _DN // 16, dstep, (neg, neg, zero, zero), unroll=8)
            accv, acci = _merge_rounds(
                (v1, v2), (t1, t2), 2,
                jnp.zeros((16,), jnp.float32), jnp.zeros((16,), jnp.int32))
            do_v[rr] = accv
            do_i[rr] = acci
            return 0

        lax.fori_loop(0, _DCH, rowbody, 0)
        pltpu.sync_copy(do_v, dv_hbm.at[pl.ds(dbase + c * _DCH, _DCH)])
        pltpu.sync_copy(do_i, di_hbm.at[pl.ds(dbase + c * _DCH, _DCH)])


def _topk_sc(x, dflat):
    xrows = x.shape[0]
    mesh = plsc.VectorSubcoreMesh(core_axis_name="c", subcore_axis_name="s")
    kern = pl.kernel(
        _sc_body,
        out_type=[
            jax.ShapeDtypeStruct((xrows, 16), jnp.float32),
            jax.ShapeDtypeStruct((xrows, 16), jnp.int32),
            jax.ShapeDtypeStruct((_DROWS, 16), jnp.float32),
            jax.ShapeDtypeStruct((_DROWS, 16), jnp.int32),
        ],
        mesh=mesh,
        scratch_types=[
            pltpu.VMEM((_XN,), jnp.float32),
            pltpu.VMEM((_DCH, _DN), jnp.float32),
            pltpu.VMEM((_DCH, _DN), jnp.float32),
            pltpu.VMEM((_XROWS_W, 16), jnp.float32),
            pltpu.VMEM((_XROWS_W, 16), jnp.int32),
            pltpu.VMEM((_DCH, 16), jnp.float32),
            pltpu.VMEM((_DCH, 16), jnp.int32),
            pltpu.SemaphoreType.DMA,
            pltpu.SemaphoreType.DMA,
        ],
    )
    xv, xi, dv, di = kern(x, dflat)
    return xv[:, :4], xi[:, :4], dv[:, :2], di[:, :2]


_YB, _ZW, _DB = 256, 512, 8
_YS, _ZS, _DS = 16, 4, 16
_Z0, _D0 = _YS, _YS + _ZS
_STEPS = _YS + _ZS + _DS


def _topk_axis(blk, k, axis, largest=True, idx_base=0):
    idx = jax.lax.broadcasted_iota(jnp.int32, blk.shape, axis) + idx_base
    fill = _NEG_INF if largest else _POS_INF
    cur = blk
    vs, js = [], []
    for t in range(k):
        if largest:
            m = jnp.max(cur, axis=axis, keepdims=True)
        else:
            m = jnp.min(cur, axis=axis, keepdims=True)
        j = jnp.min(jnp.where(cur == m, idx, _BIG_I32), axis=axis, keepdims=True)
        vs.append(m)
        js.append(j)
        if t + 1 < k:
            cur = jnp.where(idx == j, fill, cur)
    return vs, js


def _body(y_ref, z_ref, d_ref,
          y1_ref, i1_ref, y2_ref, i2_ref,
          i3_ref, i4_ref, z1_ref, i5_ref,
          d2_ref, i8_ref,
          v_s, p_s):
    g = pl.program_id(0)

    @pl.when(g < _Z0)
    def _y_phase():
        gy = g
        blk = y_ref[...]

        (m1, m2), (j1, j2) = _topk_axis(blk, 2, 0, largest=True,
                                        idx_base=gy * _YB)

        @pl.when(gy == 0)
        def _():
            y1_ref[...] = jnp.concatenate([m1, m2], axis=0)
            i1_ref[...] = jnp.concatenate([j1, j2], axis=0)

        @pl.when(gy > 0)
        def _():
            v1 = y1_ref[0:1, :]
            v2 = y1_ref[1:2, :]
            p1 = i1_ref[0:1, :]
            p2 = i1_ref[1:2, :]
            take1 = m1 > v1
            a_v = jnp.where(take1, v1, v2)
            a_i = jnp.where(take1, p1, p2)
            b_v = jnp.where(take1, m2, m1)
            b_i = jnp.where(take1, j2, j1)
            take2 = b_v > a_v
            y1_ref[...] = jnp.concatenate(
                [jnp.where(take1, m1, v1), jnp.where(take2, b_v, a_v)], axis=0)
            i1_ref[...] = jnp.concatenate(
                [jnp.where(take1, j1, p1), jnp.where(take2, b_i, a_i)], axis=0)

        (n1, n2), (k1, k2) = _topk_axis(blk, 2, 1, largest=False)
        y2_ref[...] = jnp.concatenate([n1, n2], axis=1)
        i2_ref[...] = jnp.concatenate([k1, k2], axis=1)

    @pl.when(jnp.logical_and(g >= _Z0, g < _D0))
    def _z_phase():
        gz = g - _Z0
        blk = z_ref[...]

        _, js = _topk_axis(blk, 2, 0, largest=True)
        i3_ref[...] = jnp.concatenate(js, axis=0)

        _, js = _topk_axis(blk, 3, 1, largest=True)
        i4_ref[...] = jnp.concatenate(js, axis=1)

        lanes = jax.lax.broadcasted_iota(jnp.int32, blk.shape, 2) + gz * _ZW
        lm = jnp.max(blk, axis=2)
        lj = jnp.min(jnp.where(blk == lm[:, :, None], lanes, _BIG_I32), axis=2)

        @pl.when(gz == 0)
        def _():
            v_s[...] = lm
            p_s[...] = lj

        @pl.when(gz > 0)
        def _():
            take = lm > v_s[...]
            v_s[...] = jnp.where(take, lm, v_s[...])
            p_s[...] = jnp.where(take, lj, p_s[...])

        @pl.when(gz == _ZS - 1)
        def _():
            z1_ref[...] = v_s[...]
            i5_ref[...] = p_s[...]

    @pl.when(g >= _D0)
    def _d_phase():
        blk = d_ref[...]

        vs, js = _topk_axis(blk, 2, 1, largest=True)
        d2_ref[...] = jnp.concatenate(vs, axis=1)
        i8_ref[...] = jnp.concatenate(js, axis=1)


def _fused(y, z, d):
    yr, yn = y.shape
    a0, a1, a2 = z.shape
    db, d1, d2n = d.shape

    def ym(g):
        return (jnp.clip(g, 0, _YS - 1), 0)

    def zm(g):
        return (0, 0, jnp.clip(g - _Z0, 0, _ZS - 1))

    def dm(g):
        return (jnp.clip(g - _D0, 0, _DS - 1), 0, 0)

    return pl.pallas_call(
        _body,
        grid=(_STEPS,),
        in_specs=[
            pl.BlockSpec((_YB, yn), ym),
            pl.BlockSpec((a0, a1, _ZW), zm),
            pl.BlockSpec((_DB, d1, d2n), dm),
        ],
        out_specs=[
            pl.BlockSpec((2, yn), lambda g: (0, 0)),
            pl.BlockSpec((2, yn), lambda g: (0, 0)),
            pl.BlockSpec((_YB, 2), ym),
            pl.BlockSpec((_YB, 2), ym),
            pl.BlockSpec((2, a1, _ZW), zm),
            pl.BlockSpec((a0, 3, _ZW), zm),
            pl.BlockSpec((a0, a1), lambda g: (0, 0)),
            pl.BlockSpec((a0, a1), lambda g: (0, 0)),
            pl.BlockSpec((_DB, 2, d2n), dm),
            pl.BlockSpec((_DB, 2, d2n), dm),
        ],
        out_shape=[
            jax.ShapeDtypeStruct((2, yn), jnp.float32),
            jax.ShapeDtypeStruct((2, yn), jnp.int32),
            jax.ShapeDtypeStruct((yr, 2), jnp.float32),
            jax.ShapeDtypeStruct((yr, 2), jnp.int32),
            jax.ShapeDtypeStruct((2, a1, a2), jnp.int32),
            jax.ShapeDtypeStruct((a0, 3, a2), jnp.int32),
            jax.ShapeDtypeStruct((a0, a1), jnp.float32),
            jax.ShapeDtypeStruct((a0, a1), jnp.int32),
            jax.ShapeDtypeStruct((db, 2, d2n), jnp.float32),
            jax.ShapeDtypeStruct((db, 2, d2n), jnp.int32),
        ],
        scratch_shapes=[
            pltpu.VMEM((a0, a1), jnp.float32),
            pltpu.VMEM((a0, a1), jnp.int32),
        ],
    )(y, z, d)


def kernel(x, y, z, d):
    b0, b1, a1, a2 = d.shape
    x0, i0, d3, i9 = _topk_sc(x, d.reshape(b0 * b1 * a1, a2))

    (y1, i1, y2, i2, i3, i4, z1, i5,
     d2, i8) = _fused(y, z, d.reshape(b0 * b1, a1, a2))
    z1 = z1[:, :, None]
    i5 = i5[:, :, None]
    d2 = d2.reshape(b0, b1, 2, a2)
    i8 = i8.reshape(b0, b1, 2, a2)
    d3 = d3.reshape(b0, b1, a1, 2)
    i9 = i9.reshape(b0, b1, a1, 2)
    return (x0, y1, y2, i0, i1, i2, z1, i3, i4, i5, d2, d3, i8, i9)
